# baseline (speedup 1.0000x reference)
import functools

import jax
import jax.numpy as jnp
from jax import lax
from jax.experimental import pallas as pl
from jax.experimental.pallas import tpu as pltpu
from jax.experimental.pallas import tpu_sc as plsc

NC, NS, L = 2, 16, 16
NW = NC * NS

B, S = 16384, 200
V, D = 10, 12
TJ = S // 8
TI_W = 4
IW = TI_W * 128
OBUF = D * TI_W * 8 * 128
OUT_FLAT = D * TJ * (B // 128) * 8 * 128

_mesh = plsc.VectorSubcoreMesh(core_axis_name="c", subcore_axis_name="s")


@functools.partial(
    pl.kernel,
    out_type=jax.ShapeDtypeStruct((OUT_FLAT,), jnp.float32),
    mesh=_mesh,
    compiler_params=pltpu.CompilerParams(needs_layout_passes=False),
    scratch_types=[
        pltpu.VMEM((V * D * L,), jnp.float32),
        pltpu.VMEM((8, IW), jnp.int32),
        pltpu.VMEM((8, IW), jnp.int32),
        pltpu.VMEM((OBUF,), jnp.float32),
        pltpu.VMEM((OBUF,), jnp.float32),
        pltpu.SemaphoreType.DMA,
        pltpu.SemaphoreType.DMA,
        pltpu.SemaphoreType.DMA,
        pltpu.SemaphoreType.DMA,
    ],
)
def _embed(xt_hbm, w_hbm, out_hbm,
           w_v, xb0, xb1, ob0, ob1, xs0, xs1, os0, os1):
    wid = lax.axis_index("s") * NC + lax.axis_index("c")
    i0 = wid * IW
    pltpu.sync_copy(w_hbm, w_v)

    def start_x(u, xb, xs):
        pltpu.async_copy(
            xt_hbm.at[pl.ds(u * 8, 8), pl.ds(i0, IW)], xb, xs)

    def wait_x(xb, xs):
        pltpu.make_async_copy(
            xt_hbm.at[pl.ds(0, 8), pl.ds(0, IW)], xb, xs).wait()

    def start_out(u, ob, os):
        for d in range(D):
            off = ((d * TJ + u) * (B // 128) + wid * TI_W) * 1024
            pltpu.async_copy(
                ob.at[pl.ds(d * TI_W * 1024, TI_W * 1024)],
                out_hbm.at[pl.ds(off, TI_W * 1024)], os)

    def drain_out(ob, os):
        pltpu.make_async_copy(ob, out_hbm.at[pl.ds(0, OBUF)], os).wait()

    lane = lax.iota(jnp.int32, L)
    diota = [d * L + lane for d in range(D)]

    def compute(xb, ob):
        @plsc.parallel_loop(0, TI_W * 8, unroll=8)
        def _(ig):
            dyn = (ig // 8) * 1024 + (ig % 8) * 16
            xvs = [xb[jl, pl.ds(ig * 16, 16)] * (D * L) for jl in range(8)]
            for jl in range(8):
                for d in range(D):
                    vals = plsc.load_gather(w_v, [xvs[jl] + diota[d]])
                    ob[pl.ds(dyn + (d * TI_W * 8 + jl) * 128, 16)] = vals

    def unit(u, xb, xs, ob, os):
        wait_x(xb, xs)

        @pl.when(u >= 2)
        def _():
            drain_out(ob, os)

        compute(xb, ob)
        start_out(u, ob, os)

        @pl.when(u + 2 < TJ)
        def _():
            start_x(u + 2, xb, xs)

    start_x(0, xb0, xs0)
    start_x(1, xb1, xs1)

    def pair(k, c):
        u = k * 2
        unit(u, xb0, xs0, ob0, os0)
        unit(u + 1, xb1, xs1, ob1, os1)
        return c

    lax.fori_loop(0, (TJ - 1) // 2, pair, 0)
    unit(jnp.int32(TJ - 1), xb0, xs0, ob0, os0)
    drain_out(ob1, os1)
    drain_out(ob0, os0)


def kernel(x, weight):
    xt = jnp.swapaxes(x, 0, 1).astype(jnp.int32)
    wrep = jnp.tile(
        weight.astype(jnp.float32).reshape(-1)[:, None], (1, L)).reshape(-1)
    out = _embed(xt, wrep)
    f5 = out.reshape(D, S // 8, B // 128, 8, 128)
    return f5.transpose(2, 4, 1, 3, 0).reshape(B, S, D)


# speedup vs baseline: 1.9549x; 1.9549x over previous
import functools

import jax
import jax.numpy as jnp
from jax import lax
from jax.experimental import pallas as pl
from jax.experimental.pallas import tpu as pltpu
from jax.experimental.pallas import tpu_sc as plsc

NC, NS, L = 2, 16, 16
NW = NC * NS

B, S = 16384, 200
V, D = 10, 12
TJ = S // 8
TI_W = 4
IW = TI_W * 128
OBUF = D * TI_W * 8 * 128
OUT_FLAT = D * TJ * (B // 128) * 8 * 128

_mesh = plsc.VectorSubcoreMesh(core_axis_name="c", subcore_axis_name="s")


@functools.partial(
    pl.kernel,
    out_type=jax.ShapeDtypeStruct((OUT_FLAT,), jnp.float32),
    mesh=_mesh,
    compiler_params=pltpu.CompilerParams(needs_layout_passes=False),
    scratch_types=[
        pltpu.VMEM((V * D * L,), jnp.float32),
        pltpu.VMEM((8, IW), jnp.int32),
        pltpu.VMEM((8, IW), jnp.int32),
        pltpu.VMEM((OBUF,), jnp.float32),
        pltpu.VMEM((OBUF,), jnp.float32),
        pltpu.SemaphoreType.DMA,
        pltpu.SemaphoreType.DMA,
        pltpu.SemaphoreType.DMA,
        pltpu.SemaphoreType.DMA,
    ],
)
def _embed(xt_hbm, w_hbm, out_hbm,
           w_v, xb0, xb1, ob0, ob1, xs0, xs1, os0, os1):
    wid = lax.axis_index("s") * NC + lax.axis_index("c")
    i0 = wid * IW
    pltpu.sync_copy(w_hbm, w_v)

    def start_x(u, xb, xs):
        pltpu.async_copy(
            xt_hbm.at[pl.ds(u * 8, 8), pl.ds(i0, IW)], xb, xs)

    def wait_x(xb, xs):
        pltpu.make_async_copy(
            xt_hbm.at[pl.ds(0, 8), pl.ds(0, IW)], xb, xs).wait()

    def start_out(u, ob, os):
        for d in range(D):
            off = ((d * TJ + u) * (B // 128) + wid * TI_W) * 1024
            pltpu.async_copy(
                ob.at[pl.ds(d * TI_W * 1024, TI_W * 1024)],
                out_hbm.at[pl.ds(off, TI_W * 1024)], os)

    def drain_out(ob, os):
        pltpu.make_async_copy(ob, out_hbm.at[pl.ds(0, OBUF)], os).wait()

    lane = lax.iota(jnp.int32, L)
    w_d = [w_v.at[pl.ds(d * L, V * D * L - (D - 1) * L)] for d in range(D)]

    def compute(xb, ob):
        @plsc.parallel_loop(0, TI_W * 8, unroll=4)
        def _(ig):
            dyn = (ig // 8) * 1024 + (ig % 8) * 16
            xvl = [xb[jl, pl.ds(ig * 16, 16)] * (D * L) + lane
                   for jl in range(8)]
            for jl in range(8):
                for d in range(D):
                    vals = plsc.load_gather(w_d[d], [xvl[jl]])
                    ob[pl.ds(dyn + (d * TI_W * 8 + jl) * 128, 16)] = vals

    def unit(u, xb, xs, ob, os):
        wait_x(xb, xs)

        @pl.when(u >= 2)
        def _():
            drain_out(ob, os)

        compute(xb, ob)
        start_out(u, ob, os)

        @pl.when(u + 2 < TJ)
        def _():
            start_x(u + 2, xb, xs)

    start_x(0, xb0, xs0)
    start_x(1, xb1, xs1)

    def pair(k, c):
        u = k * 2
        unit(u, xb0, xs0, ob0, os0)
        unit(u + 1, xb1, xs1, ob1, os1)
        return c

    lax.fori_loop(0, (TJ - 1) // 2, pair, 0)
    unit(jnp.int32(TJ - 1), xb0, xs0, ob0, os0)
    drain_out(ob1, os1)
    drain_out(ob0, os0)


def kernel(x, weight):
    xt = jnp.swapaxes(x, 0, 1).astype(jnp.int32)
    wrep = jnp.tile(
        weight.astype(jnp.float32).reshape(-1)[:, None], (1, L)).reshape(-1)
    out = _embed(xt, wrep)
    f5 = out.reshape(D, S // 8, B // 128, 8, 128)
    return f5.transpose(2, 4, 1, 3, 0).reshape(B, S, D)


# speedup vs baseline: 3.8421x; 1.9654x over previous
import functools

import jax
import jax.numpy as jnp
from jax import lax
from jax.experimental import pallas as pl
from jax.experimental.pallas import tpu as pltpu
from jax.experimental.pallas import tpu_sc as plsc

NC, NS, L = 2, 16, 16
NW = NC * NS

B, S = 16384, 200
V, D = 10, 12
TJ = S // 8
TI_W = 4
IW = TI_W * 128
OBUF = D * TI_W * 8 * 128
OUT_FLAT = D * TJ * (B // 128) * 8 * 128

_mesh = plsc.VectorSubcoreMesh(core_axis_name="c", subcore_axis_name="s")


@functools.partial(
    pl.kernel,
    out_type=jax.ShapeDtypeStruct((OUT_FLAT,), jnp.float32),
    mesh=_mesh,
    compiler_params=pltpu.CompilerParams(needs_layout_passes=False),
    scratch_types=[
        pltpu.VMEM((D * L,), jnp.float32),
        pltpu.VMEM((8, IW), jnp.int32),
        pltpu.VMEM((8, IW), jnp.int32),
        pltpu.VMEM((OBUF,), jnp.float32),
        pltpu.VMEM((OBUF,), jnp.float32),
        pltpu.SemaphoreType.DMA,
        pltpu.SemaphoreType.DMA,
        pltpu.SemaphoreType.DMA,
        pltpu.SemaphoreType.DMA,
    ],
)
def _embed(xt_hbm, w_hbm, out_hbm,
           w_v, xb0, xb1, ob0, ob1, xs0, xs1, os0, os1):
    wid = lax.axis_index("s") * NC + lax.axis_index("c")
    i0 = wid * IW
    pltpu.sync_copy(w_hbm, w_v)

    def start_x(u, xb, xs):
        pltpu.async_copy(
            xt_hbm.at[pl.ds(u * 8, 8), pl.ds(i0, IW)], xb, xs)

    def wait_x(xb, xs):
        pltpu.make_async_copy(
            xt_hbm.at[pl.ds(0, 8), pl.ds(0, IW)], xb, xs).wait()

    def start_out(u, ob, os):
        for d in range(D):
            off = ((d * TJ + u) * (B // 128) + wid * TI_W) * 1024
            pltpu.async_copy(
                ob.at[pl.ds(d * TI_W * 1024, TI_W * 1024)],
                out_hbm.at[pl.ds(off, TI_W * 1024)], os)

    def drain_out(ob, os):
        pltpu.make_async_copy(ob, out_hbm.at[pl.ds(0, OBUF)], os).wait()

    cols = [w_v[pl.ds(d * L, L)] for d in range(D)]

    def compute(xb, ob):
        @plsc.parallel_loop(0, TI_W * 8, unroll=4)
        def _(ig):
            dyn = (ig // 8) * 1024 + (ig % 8) * 16
            xvs = [xb[jl, pl.ds(ig * 16, 16)] for jl in range(8)]
            for jl in range(8):
                for d in range(D):
                    vals = cols[d].at[xvs[jl]].get(
                        mode="promise_in_bounds")
                    ob[pl.ds(dyn + (d * TI_W * 8 + jl) * 128, 16)] = vals

    def unit(u, xb, xs, ob, os):
        wait_x(xb, xs)

        @pl.when(u >= 2)
        def _():
            drain_out(ob, os)

        compute(xb, ob)
        start_out(u, ob, os)

        @pl.when(u + 2 < TJ)
        def _():
            start_x(u + 2, xb, xs)

    start_x(0, xb0, xs0)
    start_x(1, xb1, xs1)

    def pair(k, c):
        u = k * 2
        unit(u, xb0, xs0, ob0, os0)
        unit(u + 1, xb1, xs1, ob1, os1)
        return c

    lax.fori_loop(0, (TJ - 1) // 2, pair, 0)
    unit(jnp.int32(TJ - 1), xb0, xs0, ob0, os0)
    drain_out(ob1, os1)
    drain_out(ob0, os0)


def kernel(x, weight):
    xt = jnp.swapaxes(x, 0, 1).astype(jnp.int32)
    wcols = jnp.pad(
        weight.astype(jnp.float32).T, ((0, 0), (0, L - V))).reshape(-1)
    out = _embed(xt, wcols)
    f5 = out.reshape(D, S // 8, B // 128, 8, 128)
    return f5.transpose(2, 4, 1, 3, 0).reshape(B, S, D)
